# Initial kernel scaffold; baseline (speedup 1.0000x reference)
import jax
import jax.numpy as jnp
from jax.experimental import pallas as pl


def kernel(atomic_energy, image_idx, n_atoms):
    raise NotImplementedError("write your pallas kernel here")


# speedup vs baseline: 10.6716x; 10.6716x over previous
import functools

import jax
import jax.numpy as jnp
from jax import lax
from jax.experimental import pallas as pl
from jax.experimental.pallas import tpu as pltpu
from jax.experimental.pallas import tpu_sc as plsc

_NC = 2
_NS = 16
_NW = _NC * _NS


def _sc_segment_partials(energy, idx, zeros, num_images, chunk):
    mesh = plsc.VectorSubcoreMesh(core_axis_name="c", subcore_axis_name="s")

    @functools.partial(
        pl.kernel,
        mesh=mesh,
        out_type=jax.ShapeDtypeStruct((_NC, 2, num_images), jnp.float32),
        scratch_types=[
            pltpu.VMEM((chunk,), jnp.float32),
            pltpu.VMEM((chunk,), jnp.int32),
            pltpu.VMEM_SHARED((num_images,), jnp.float32),
            pltpu.VMEM_SHARED((num_images,), jnp.float32),
        ],
    )
    def k(e_hbm, i_hbm, z_hbm, out_hbm, e_v, i_v, acc_even, acc_odd):
        c = lax.axis_index("c")
        s = lax.axis_index("s")
        base = (c * _NS + s) * chunk
        parity = s % 2

        @pl.when(s == 0)
        def _():
            pltpu.sync_copy(z_hbm, acc_even)

        @pl.when(s == 1)
        def _():
            pltpu.sync_copy(z_hbm, acc_odd)

        plsc.subcore_barrier()

        pltpu.sync_copy(e_hbm.at[pl.ds(base, chunk)], e_v)
        pltpu.sync_copy(i_hbm.at[pl.ds(base, chunk)], i_v)

        @pl.when(parity == 0)
        def _():
            pltpu.sync_copy(e_v, acc_even.at[i_v], add=True)

        @pl.when(parity == 1)
        def _():
            pltpu.sync_copy(e_v, acc_odd.at[i_v], add=True)

        plsc.subcore_barrier()

        @pl.when(s == 0)
        def _():
            pltpu.sync_copy(acc_even, out_hbm.at[c, 0])

        @pl.when(s == 1)
        def _():
            pltpu.sync_copy(acc_odd, out_hbm.at[c, 1])

    return k(energy, idx, zeros)


def _tc_merge(partials, num_images):
    def body(p_ref, o_ref):
        o_ref[...] = (p_ref[0, 0] + p_ref[0, 1]) + (p_ref[1, 0] + p_ref[1, 1])

    return pl.pallas_call(
        body,
        out_shape=jax.ShapeDtypeStruct((num_images,), jnp.float32),
    )(partials)


def kernel(atomic_energy, image_idx, n_atoms):
    n = atomic_energy.shape[0]
    num_images = n_atoms.shape[0]
    assert n % _NW == 0
    chunk = n // _NW

    idx32 = image_idx.astype(jnp.int32)
    energy = atomic_energy.astype(jnp.float32)
    zeros = jnp.zeros((num_images,), jnp.float32)

    partials = _sc_segment_partials(energy, idx32, zeros, num_images, chunk)
    return _tc_merge(partials, num_images)


# speedup vs baseline: 19.1695x; 1.7963x over previous
import functools

import jax
import jax.numpy as jnp
from jax import lax
from jax.experimental import pallas as pl
from jax.experimental.pallas import tpu as pltpu
from jax.experimental.pallas import tpu_sc as plsc

_NC = 2
_NS = 16
_NW = _NC * _NS
_L = 16
_NSUB = 5


def _sc_segment_partials(energy, idx, num_images, chunk):
    mesh = plsc.VectorSubcoreMesh(core_axis_name="c", subcore_axis_name="s")
    sub = chunk // _NSUB
    assert sub % _L == 0 and chunk == sub * _NSUB

    @functools.partial(
        pl.kernel,
        mesh=mesh,
        compiler_params=pltpu.CompilerParams(needs_layout_passes=False),
        out_type=jax.ShapeDtypeStruct((_NW, num_images), jnp.float32),
        scratch_types=[
            pltpu.VMEM((_L * num_images,), jnp.float32),
            pltpu.VMEM((num_images,), jnp.float32),
            pltpu.VMEM((sub,), jnp.float32),
            pltpu.VMEM((sub,), jnp.float32),
            pltpu.VMEM((sub,), jnp.int32),
            pltpu.VMEM((sub,), jnp.int32),
            pltpu.SemaphoreType.DMA,
            pltpu.SemaphoreType.DMA,
        ],
    )
    def k(e_hbm, i_hbm, out_hbm, acc, red, e_v0, e_v1, i_v0, i_v1, sem0, sem1):
        c = lax.axis_index("c")
        s = lax.axis_index("s")
        wid = c * _NS + s
        base = wid * chunk
        e_bufs = (e_v0, e_v1)
        i_bufs = (i_v0, i_v1)
        sems = (sem0, sem1)

        def start(j):
            b = j % 2
            ce = pltpu.async_copy(
                e_hbm.at[pl.ds(base + j * sub, sub)], e_bufs[b], sems[b])
            ci = pltpu.async_copy(
                i_hbm.at[pl.ds(base + j * sub, sub)], i_bufs[b], sems[b])
            return ce, ci

        pend = start(0)

        zero = jnp.zeros((_L,), jnp.float32)

        def zb(j, _):
            acc[pl.ds(j * _L, _L)] = zero
            return _

        lax.fori_loop(0, (_L * num_images) // _L, zb, 0)

        lane_off = lax.iota(jnp.int32, _L) * num_images

        for j in range(_NSUB):
            b = j % 2
            for cc in pend:
                cc.wait()
            if j + 1 < _NSUB:
                pend = start(j + 1)

            def body(v, _):
                e16 = e_bufs[b][pl.ds(v * _L, _L)]
                i16 = i_bufs[b][pl.ds(v * _L, _L)]
                plsc.addupdate_scatter(acc, [i16 + lane_off], e16)
                return _

            lax.fori_loop(0, sub // _L, body, 0)

        def rbody(jv, _):
            t = acc[pl.ds(jv * _L, _L)]
            for r in range(1, _L):
                t = t + acc[pl.ds(r * num_images + jv * _L, _L)]
            red[pl.ds(jv * _L, _L)] = t
            return _

        lax.fori_loop(0, num_images // _L, rbody, 0)

        pltpu.sync_copy(red, out_hbm.at[wid])

    return k(energy, idx)


def _tc_merge(partials, num_images):
    def body(p_ref, o_ref):
        o_ref[...] = jnp.sum(p_ref[...], axis=0)

    return pl.pallas_call(
        body,
        out_shape=jax.ShapeDtypeStruct((num_images,), jnp.float32),
    )(partials)


def kernel(atomic_energy, image_idx, n_atoms):
    n = atomic_energy.shape[0]
    num_images = n_atoms.shape[0]
    assert n % _NW == 0
    chunk = n // _NW

    idx32 = image_idx.astype(jnp.int32)
    energy = atomic_energy.astype(jnp.float32)

    partials = _sc_segment_partials(energy, idx32, num_images, chunk)
    return _tc_merge(partials, num_images)


# speedup vs baseline: 20.8175x; 1.0860x over previous
import functools

import jax
import jax.numpy as jnp
from jax import lax
from jax.experimental import pallas as pl
from jax.experimental.pallas import tpu as pltpu
from jax.experimental.pallas import tpu_sc as plsc

_NC = 2
_NS = 16
_NW = _NC * _NS
_L = 16
_NSUB = 5
_UNROLL = 5


def _sc_segment_partials(energy, idx, num_images, chunk):
    mesh = plsc.VectorSubcoreMesh(core_axis_name="c", subcore_axis_name="s")
    sub = chunk // _NSUB
    nvec = sub // _L
    assert chunk == sub * _NSUB and sub % _L == 0 and nvec % _UNROLL == 0
    acc_len = _L * num_images

    @functools.partial(
        pl.kernel,
        mesh=mesh,
        compiler_params=pltpu.CompilerParams(needs_layout_passes=False),
        out_type=jax.ShapeDtypeStruct((_NW, acc_len), jnp.float32),
        scratch_types=[
            pltpu.VMEM((acc_len,), jnp.float32),
            pltpu.VMEM((sub,), jnp.float32),
            pltpu.VMEM((sub,), jnp.float32),
            pltpu.VMEM((sub,), jnp.int32),
            pltpu.VMEM((sub,), jnp.int32),
            pltpu.SemaphoreType.DMA,
            pltpu.SemaphoreType.DMA,
        ],
    )
    def k(e_hbm, i_hbm, out_hbm, acc, e_v0, e_v1, i_v0, i_v1, sem0, sem1):
        c = lax.axis_index("c")
        s = lax.axis_index("s")
        wid = c * _NS + s
        base = wid * chunk
        e_bufs = (e_v0, e_v1)
        i_bufs = (i_v0, i_v1)
        sems = (sem0, sem1)

        def start(j):
            b = j % 2
            ce = pltpu.async_copy(
                e_hbm.at[pl.ds(base + j * sub, sub)], e_bufs[b], sems[b])
            ci = pltpu.async_copy(
                i_hbm.at[pl.ds(base + j * sub, sub)], i_bufs[b], sems[b])
            return ce, ci

        pend = start(0)

        zero = jnp.zeros((_L,), jnp.float32)

        def zb(j, _):
            for u in range(16):
                acc[pl.ds((j * 16 + u) * _L, _L)] = zero
            return _

        lax.fori_loop(0, acc_len // (_L * 16), zb, 0)

        lane_off = lax.iota(jnp.int32, _L) * num_images

        for j in range(_NSUB):
            b = j % 2
            for cc in pend:
                cc.wait()
            if j + 1 < _NSUB:
                pend = start(j + 1)

            def body(v, _):
                for u in range(_UNROLL):
                    sl = pl.ds((v * _UNROLL + u) * _L, _L)
                    e16 = e_bufs[b][sl]
                    i16 = i_bufs[b][sl]
                    plsc.addupdate_scatter(acc, [i16 + lane_off], e16)
                return _

            lax.fori_loop(0, nvec // _UNROLL, body, 0)

        pltpu.sync_copy(acc, out_hbm.at[wid])

    return k(energy, idx)


def _tc_merge(partials, num_images):
    def body(p_ref, o_ref):
        o_ref[...] = jnp.sum(p_ref[...], axis=0)

    return pl.pallas_call(
        body,
        out_shape=jax.ShapeDtypeStruct((num_images,), jnp.float32),
    )(partials)


def kernel(atomic_energy, image_idx, n_atoms):
    n = atomic_energy.shape[0]
    num_images = n_atoms.shape[0]
    assert n % _NW == 0
    chunk = n // _NW

    idx32 = image_idx.astype(jnp.int32)
    energy = atomic_energy.astype(jnp.float32)

    partials = _sc_segment_partials(energy, idx32, num_images, chunk)
    lanes = partials.reshape(_NW * _L, num_images)
    return _tc_merge(lanes, num_images)


# speedup vs baseline: 29.6083x; 1.4223x over previous
import functools

import jax
import jax.numpy as jnp
from jax import lax
from jax.experimental import pallas as pl
from jax.experimental.pallas import tpu as pltpu
from jax.experimental.pallas import tpu_sc as plsc

_NC = 2
_NS = 16
_NW = _NC * _NS
_L = 16
_NSUB = 5
_UNROLL = 5


def _sc_segment_partials(energy, idx, num_images, chunk):
    mesh = plsc.VectorSubcoreMesh(core_axis_name="c", subcore_axis_name="s")
    sub = chunk // _NSUB
    nvec = sub // _L
    assert chunk == sub * _NSUB and sub % _L == 0 and nvec % _UNROLL == 0
    acc_len = _L * num_images

    @functools.partial(
        pl.kernel,
        mesh=mesh,
        compiler_params=pltpu.CompilerParams(needs_layout_passes=False),
        out_type=jax.ShapeDtypeStruct((_NW, num_images), jnp.float32),
        scratch_types=[
            pltpu.VMEM((acc_len,), jnp.float32),
            pltpu.VMEM((num_images,), jnp.float32),
            pltpu.VMEM((sub,), jnp.float32),
            pltpu.VMEM((sub,), jnp.float32),
            pltpu.VMEM((sub,), jnp.int32),
            pltpu.VMEM((sub,), jnp.int32),
            pltpu.SemaphoreType.DMA,
            pltpu.SemaphoreType.DMA,
        ],
    )
    def k(e_hbm, i_hbm, out_hbm, acc, red, e_v0, e_v1, i_v0, i_v1, sem0, sem1):
        c = lax.axis_index("c")
        s = lax.axis_index("s")
        wid = c * _NS + s
        base = wid * chunk
        e_bufs = (e_v0, e_v1)
        i_bufs = (i_v0, i_v1)
        sems = (sem0, sem1)

        def start(j):
            b = j % 2
            ce = pltpu.async_copy(
                e_hbm.at[pl.ds(base + j * sub, sub)], e_bufs[b], sems[b])
            ci = pltpu.async_copy(
                i_hbm.at[pl.ds(base + j * sub, sub)], i_bufs[b], sems[b])
            return ce, ci

        pend = start(0)

        zero = jnp.zeros((_L,), jnp.float32)

        @plsc.parallel_loop(0, acc_len // _L, unroll=8)
        def _(jv):
            acc[pl.ds(jv * _L, _L)] = zero

        lane_off = lax.iota(jnp.int32, _L) * num_images

        for j in range(_NSUB):
            b = j % 2
            for cc in pend:
                cc.wait()
            if j + 1 < _NSUB:
                pend = start(j + 1)
            eb = e_bufs[b]
            ib = i_bufs[b]

            @plsc.parallel_loop(0, nvec, unroll=_UNROLL)
            def _(v):
                sl = pl.ds(v * _L, _L)
                plsc.addupdate_scatter(acc, [ib[sl] + lane_off], eb[sl])

        @plsc.parallel_loop(0, num_images // _L, unroll=2)
        def _(jv):
            t = acc[pl.ds(jv * _L, _L)]
            for r in range(1, _L):
                t = t + acc[pl.ds(r * num_images + jv * _L, _L)]
            red[pl.ds(jv * _L, _L)] = t

        pltpu.sync_copy(red, out_hbm.at[wid])

    return k(energy, idx)


def _tc_merge(partials, num_images):
    def body(p_ref, o_ref):
        o_ref[...] = jnp.sum(p_ref[...], axis=0)

    return pl.pallas_call(
        body,
        out_shape=jax.ShapeDtypeStruct((num_images,), jnp.float32),
    )(partials)


def kernel(atomic_energy, image_idx, n_atoms):
    n = atomic_energy.shape[0]
    num_images = n_atoms.shape[0]
    assert n % _NW == 0
    chunk = n // _NW

    idx32 = image_idx.astype(jnp.int32)
    energy = atomic_energy.astype(jnp.float32)

    partials = _sc_segment_partials(energy, idx32, num_images, chunk)
    return _tc_merge(partials, num_images)


# speedup vs baseline: 31.0103x; 1.0474x over previous
import functools

import jax
import jax.numpy as jnp
from jax import lax
from jax.experimental import pallas as pl
from jax.experimental.pallas import tpu as pltpu
from jax.experimental.pallas import tpu_sc as plsc

_NC = 2
_NS = 16
_NW = _NC * _NS
_L = 16
_NSUB = 5
_UNROLL = 5


def _sc_segment_partials(energy, idx, num_images, chunk):
    mesh = plsc.VectorSubcoreMesh(core_axis_name="c", subcore_axis_name="s")
    sub = chunk // _NSUB
    nvec = sub // _L
    assert chunk == sub * _NSUB and sub % _L == 0 and nvec % _UNROLL == 0
    acc_len = _L * num_images

    @functools.partial(
        pl.kernel,
        mesh=mesh,
        compiler_params=pltpu.CompilerParams(needs_layout_passes=False),
        out_type=jax.ShapeDtypeStruct((_NW, num_images), jnp.float32),
        scratch_types=[
            pltpu.VMEM((acc_len,), jnp.float32),
            pltpu.VMEM((num_images,), jnp.float32),
            pltpu.VMEM((sub,), jnp.float32),
            pltpu.VMEM((sub,), jnp.float32),
            pltpu.VMEM((sub,), jnp.int32),
            pltpu.VMEM((sub,), jnp.int32),
            pltpu.VMEM((_L,), jnp.int32),
            pltpu.SemaphoreType.DMA,
            pltpu.SemaphoreType.DMA,
        ],
    )
    def k(e_hbm, i_hbm, out_hbm, acc, red, e_v0, e_v1, i_v0, i_v1, tail_v,
          sem0, sem1):
        c = lax.axis_index("c")
        s = lax.axis_index("s")
        wid = c * _NS + s
        base = wid * chunk
        e_bufs = (e_v0, e_v1)
        i_bufs = (i_v0, i_v1)
        sems = (sem0, sem1)

        def start(j):
            b = j % 2
            ce = pltpu.async_copy(
                e_hbm.at[pl.ds(base + j * sub, sub)], e_bufs[b], sems[b])
            ci = pltpu.async_copy(
                i_hbm.at[pl.ds(base + j * sub, sub)], i_bufs[b], sems[b])
            return ce, ci

        pend = start(0)
        ctail = pltpu.async_copy(
            i_hbm.at[pl.ds(base + chunk - _L, _L)], tail_v, sems[0])

        zero = jnp.zeros((_L,), jnp.float32)

        @plsc.parallel_loop(0, num_images // _L, unroll=8)
        def _(jv):
            red[pl.ds(jv * _L, _L)] = zero

        ctail.wait()
        for cc in pend:
            cc.wait()
        pend = ()

        lo = lax.reduce_min(i_bufs[0][pl.ds(0, _L)], (0,))
        hi = lax.reduce_max(tail_v[...], (0,))
        loa = lax.shift_left(lax.shift_right_logical(lo, 4), 4)
        nvz = lax.shift_right_logical(hi - loa, 4) + 1

        @plsc.parallel_loop(0, nvz)
        def _(jv):
            for r in range(_L):
                acc[pl.ds(r * num_images + loa + jv * _L, _L)] = zero

        lane_off = lax.iota(jnp.int32, _L) * num_images

        for j in range(_NSUB):
            b = j % 2
            for cc in pend:
                cc.wait()
            if j + 1 < _NSUB:
                pend = start(j + 1)
            else:
                pend = ()
            eb = e_bufs[b]
            ib = i_bufs[b]

            @plsc.parallel_loop(0, nvec, unroll=_UNROLL)
            def _(v):
                sl = pl.ds(v * _L, _L)
                plsc.addupdate_scatter(acc, [ib[sl] + lane_off], eb[sl])

        @plsc.parallel_loop(0, nvz)
        def _(jv):
            t = acc[pl.ds(loa + jv * _L, _L)]
            for r in range(1, _L):
                t = t + acc[pl.ds(r * num_images + loa + jv * _L, _L)]
            red[pl.ds(loa + jv * _L, _L)] = t

        pltpu.sync_copy(red, out_hbm.at[wid])

    return k(energy, idx)


def _tc_merge(partials, num_images):
    def body(p_ref, o_ref):
        o_ref[...] = jnp.sum(p_ref[...], axis=0)

    return pl.pallas_call(
        body,
        out_shape=jax.ShapeDtypeStruct((num_images,), jnp.float32),
    )(partials)


def kernel(atomic_energy, image_idx, n_atoms):
    n = atomic_energy.shape[0]
    num_images = n_atoms.shape[0]
    assert n % _NW == 0
    chunk = n // _NW

    idx32 = image_idx.astype(jnp.int32)
    energy = atomic_energy.astype(jnp.float32)

    partials = _sc_segment_partials(energy, idx32, num_images, chunk)
    return _tc_merge(partials, num_images)


# speedup vs baseline: 31.0166x; 1.0002x over previous
import functools

import jax
import jax.numpy as jnp
from jax import lax
from jax.experimental import pallas as pl
from jax.experimental.pallas import tpu as pltpu
from jax.experimental.pallas import tpu_sc as plsc

_NC = 2
_NS = 16
_NW = _NC * _NS
_L = 16
_NSUB = 5
_UNROLL = 5


def _sc_segment_partials(energy, idx, num_images, chunk):
    mesh = plsc.VectorSubcoreMesh(core_axis_name="c", subcore_axis_name="s")
    sub = chunk // _NSUB
    nvec = sub // _L
    assert chunk == sub * _NSUB and sub % _L == 0 and nvec % _UNROLL == 0
    acc_len = _L * num_images

    @functools.partial(
        pl.kernel,
        mesh=mesh,
        compiler_params=pltpu.CompilerParams(needs_layout_passes=False),
        out_type=jax.ShapeDtypeStruct((_NW, num_images), jnp.float32),
        scratch_types=[
            pltpu.VMEM((acc_len,), jnp.float32),
            pltpu.VMEM((num_images,), jnp.float32),
            pltpu.VMEM((sub,), jnp.float32),
            pltpu.VMEM((sub,), jnp.float32),
            pltpu.VMEM((sub,), jnp.int32),
            pltpu.VMEM((sub,), jnp.int32),
            pltpu.VMEM((_L,), jnp.int32),
            pltpu.SemaphoreType.DMA,
            pltpu.SemaphoreType.DMA,
        ],
    )
    def k(e_hbm, i_hbm, out_hbm, acc, red, e_v0, e_v1, i_v0, i_v1, tail_v,
          sem0, sem1):
        c = lax.axis_index("c")
        s = lax.axis_index("s")
        wid = c * _NS + s
        base = wid * chunk
        e_bufs = (e_v0, e_v1)
        i_bufs = (i_v0, i_v1)
        sems = (sem0, sem1)

        def start(j):
            b = j % 2
            ce = pltpu.async_copy(
                e_hbm.at[pl.ds(base + j * sub, sub)], e_bufs[b], sems[b])
            ci = pltpu.async_copy(
                i_hbm.at[pl.ds(base + j * sub, sub)], i_bufs[b], sems[b])
            return ce, ci

        pend = start(0)
        ctail = pltpu.async_copy(
            i_hbm.at[pl.ds(base + chunk - _L, _L)], tail_v, sems[0])

        zero = jnp.zeros((_L,), jnp.float32)

        with jax.named_scope("zero_red"):
            @plsc.parallel_loop(0, num_images // _L, unroll=8)
            def _(jv):
                red[pl.ds(jv * _L, _L)] = zero

        with jax.named_scope("wait_first"):
            ctail.wait()
            for cc in pend:
                cc.wait()
            pend = ()

        lo = lax.reduce_min(i_bufs[0][pl.ds(0, _L)], (0,))
        hi = lax.reduce_max(tail_v[...], (0,))
        loa = lax.shift_left(lax.shift_right_logical(lo, 4), 4)
        nvz = lax.shift_right_logical(hi - loa, 4) + 1

        with jax.named_scope("zero_acc"):
            @plsc.parallel_loop(0, nvz)
            def _(jv):
                for r in range(_L):
                    acc[pl.ds(r * num_images + loa + jv * _L, _L)] = zero

        lane_off = lax.iota(jnp.int32, _L) * num_images

        for j in range(_NSUB):
            b = j % 2
            with jax.named_scope(f"wait_{j}"):
                for cc in pend:
                    cc.wait()
            if j + 1 < _NSUB:
                pend = start(j + 1)
            else:
                pend = ()
            eb = e_bufs[b]
            ib = i_bufs[b]

            with jax.named_scope(f"scatter_{j}"):
                @plsc.parallel_loop(0, nvec, unroll=_UNROLL)
                def _(v):
                    sl = pl.ds(v * _L, _L)
                    plsc.addupdate_scatter(acc, [ib[sl] + lane_off], eb[sl])

        with jax.named_scope("reduce"):
            @plsc.parallel_loop(0, nvz)
            def _(jv):
                t = acc[pl.ds(loa + jv * _L, _L)]
                for r in range(1, _L):
                    t = t + acc[pl.ds(r * num_images + loa + jv * _L, _L)]
                red[pl.ds(loa + jv * _L, _L)] = t

        with jax.named_scope("writeout"):
            pltpu.sync_copy(red, out_hbm.at[wid])

    return k(energy, idx)


def _tc_merge(partials, num_images):
    def body(p_ref, o_ref):
        o_ref[...] = jnp.sum(p_ref[...], axis=0)

    return pl.pallas_call(
        body,
        out_shape=jax.ShapeDtypeStruct((num_images,), jnp.float32),
    )(partials)


def kernel(atomic_energy, image_idx, n_atoms):
    n = atomic_energy.shape[0]
    num_images = n_atoms.shape[0]
    assert n % _NW == 0
    chunk = n // _NW

    idx32 = image_idx.astype(jnp.int32)
    energy = atomic_energy.astype(jnp.float32)

    partials = _sc_segment_partials(energy, idx32, num_images, chunk)
    return _tc_merge(partials, num_images)


# speedup vs baseline: 31.2681x; 1.0081x over previous
import functools

import jax
import jax.numpy as jnp
from jax import lax
from jax.experimental import pallas as pl
from jax.experimental.pallas import tpu as pltpu
from jax.experimental.pallas import tpu_sc as plsc

_NC = 2
_NS = 16
_NW = _NC * _NS
_L = 16
_NSUB = 5
_UNROLL = 5


def _sc_segment_partials(energy, idx, num_images, chunk):
    mesh = plsc.VectorSubcoreMesh(core_axis_name="c", subcore_axis_name="s")
    sub = chunk // _NSUB
    nvec = sub // _L
    assert chunk == sub * _NSUB and sub % _L == 0 and nvec % _UNROLL == 0
    acc_len = _L * num_images

    @functools.partial(
        pl.kernel,
        mesh=mesh,
        compiler_params=pltpu.CompilerParams(needs_layout_passes=False),
        out_type=jax.ShapeDtypeStruct((_NW, num_images), jnp.float32),
        scratch_types=[
            pltpu.VMEM((acc_len,), jnp.float32),
            pltpu.VMEM((num_images,), jnp.float32),
            pltpu.VMEM((sub,), jnp.float32),
            pltpu.VMEM((sub,), jnp.float32),
            pltpu.VMEM((sub,), jnp.int32),
            pltpu.VMEM((sub,), jnp.int32),
            pltpu.VMEM((_L,), jnp.int32),
            pltpu.SemaphoreType.DMA,
            pltpu.SemaphoreType.DMA,
        ],
    )
    def k(e_hbm, i_hbm, out_hbm, acc, red, e_v0, e_v1, i_v0, i_v1, tail_v,
          sem0, sem1):
        c = lax.axis_index("c")
        s = lax.axis_index("s")
        wid = c * _NS + s
        base = wid * chunk
        e_bufs = (e_v0, e_v1)
        i_bufs = (i_v0, i_v1)
        sems = (sem0, sem1)

        def start(j):
            b = j % 2
            ce = pltpu.async_copy(
                e_hbm.at[pl.ds(base + j * sub, sub)], e_bufs[b], sems[b])
            ci = pltpu.async_copy(
                i_hbm.at[pl.ds(base + j * sub, sub)], i_bufs[b], sems[b])
            return ce, ci

        pend = start(0)
        ctail = pltpu.async_copy(
            i_hbm.at[pl.ds(base + chunk - _L, _L)], tail_v, sems[0])

        zero = jnp.zeros((_L,), jnp.float32)

        with jax.named_scope("zero_red"):
            @plsc.parallel_loop(0, num_images // _L, unroll=8)
            def _(jv):
                red[pl.ds(jv * _L, _L)] = zero

        with jax.named_scope("wait_first"):
            ctail.wait()
            for cc in pend:
                cc.wait()
            pend = ()

        lo = lax.reduce_min(i_bufs[0][pl.ds(0, _L)], (0,))
        hi = lax.reduce_max(tail_v[...], (0,))
        loa = lax.shift_left(lax.shift_right_logical(lo, 4), 4)
        nvz = lax.shift_right_logical(hi - loa, 4) + 1

        with jax.named_scope("zero_acc"):
            @plsc.parallel_loop(0, nvz)
            def _(jv):
                for r in range(_L):
                    acc[pl.ds(r * num_images + loa + jv * _L, _L)] = zero

        lane_off = lax.iota(jnp.int32, _L) * num_images

        for j in range(_NSUB):
            b = j % 2
            with jax.named_scope(f"wait_{j}"):
                for cc in pend:
                    cc.wait()
            if j + 1 < _NSUB:
                pend = start(j + 1)
            else:
                pend = ()
            eb = e_bufs[b]
            ib = i_bufs[b]

            with jax.named_scope(f"scatter_{j}"):
                stride = nvec // _UNROLL

                @plsc.parallel_loop(0, stride)
                def _(v):
                    for q in range(_UNROLL):
                        sl = pl.ds((q * stride + v) * _L, _L)
                        plsc.addupdate_scatter(
                            acc, [ib[sl] + lane_off], eb[sl])

        with jax.named_scope("reduce"):
            @plsc.parallel_loop(0, nvz)
            def _(jv):
                t = acc[pl.ds(loa + jv * _L, _L)]
                for r in range(1, _L):
                    t = t + acc[pl.ds(r * num_images + loa + jv * _L, _L)]
                red[pl.ds(loa + jv * _L, _L)] = t

        with jax.named_scope("writeout"):
            pltpu.sync_copy(red, out_hbm.at[wid])

    return k(energy, idx)


def _tc_merge(partials, num_images):
    def body(p_ref, o_ref):
        o_ref[...] = jnp.sum(p_ref[...], axis=0)

    return pl.pallas_call(
        body,
        out_shape=jax.ShapeDtypeStruct((num_images,), jnp.float32),
    )(partials)


def kernel(atomic_energy, image_idx, n_atoms):
    n = atomic_energy.shape[0]
    num_images = n_atoms.shape[0]
    assert n % _NW == 0
    chunk = n // _NW

    idx32 = image_idx.astype(jnp.int32)
    energy = atomic_energy.astype(jnp.float32)

    partials = _sc_segment_partials(energy, idx32, num_images, chunk)
    return _tc_merge(partials, num_images)


# speedup vs baseline: 55.9231x; 1.7885x over previous
import functools

import jax
import jax.numpy as jnp
from jax import lax
from jax.experimental import pallas as pl
from jax.experimental.pallas import tpu as pltpu
from jax.experimental.pallas import tpu_sc as plsc

_NC = 2
_NS = 16
_NW = _NC * _NS
_L = 16
_NSUB = 5
_UNROLL = 5


def _sc_segment_partials(energy, idx, num_images, chunk):
    mesh = plsc.VectorSubcoreMesh(core_axis_name="c", subcore_axis_name="s")
    sub = chunk // _NSUB
    nvec = sub // _L
    assert chunk == sub * _NSUB and sub % _L == 0 and nvec % _UNROLL == 0
    pitch = num_images + 1
    acc_len = _L * pitch

    @functools.partial(
        pl.kernel,
        mesh=mesh,
        compiler_params=pltpu.CompilerParams(needs_layout_passes=False),
        out_type=jax.ShapeDtypeStruct((_NW, num_images), jnp.float32),
        scratch_types=[
            pltpu.VMEM((acc_len,), jnp.float32),
            pltpu.VMEM((num_images,), jnp.float32),
            pltpu.VMEM((sub,), jnp.float32),
            pltpu.VMEM((sub,), jnp.float32),
            pltpu.VMEM((sub,), jnp.int32),
            pltpu.VMEM((sub,), jnp.int32),
            pltpu.VMEM((_L,), jnp.int32),
            pltpu.SemaphoreType.DMA,
            pltpu.SemaphoreType.DMA,
        ],
    )
    def k(e_hbm, i_hbm, out_hbm, acc, red, e_v0, e_v1, i_v0, i_v1, tail_v,
          sem0, sem1):
        c = lax.axis_index("c")
        s = lax.axis_index("s")
        wid = c * _NS + s
        base = wid * chunk
        e_bufs = (e_v0, e_v1)
        i_bufs = (i_v0, i_v1)
        sems = (sem0, sem1)

        def start(j):
            b = j % 2
            ce = pltpu.async_copy(
                e_hbm.at[pl.ds(base + j * sub, sub)], e_bufs[b], sems[b])
            ci = pltpu.async_copy(
                i_hbm.at[pl.ds(base + j * sub, sub)], i_bufs[b], sems[b])
            return ce, ci

        pend = start(0)
        ctail = pltpu.async_copy(
            i_hbm.at[pl.ds(base + chunk - _L, _L)], tail_v, sems[0])

        zero = jnp.zeros((_L,), jnp.float32)

        with jax.named_scope("zero_red"):
            @plsc.parallel_loop(0, num_images // _L, unroll=8)
            def _(jv):
                red[pl.ds(jv * _L, _L)] = zero

        with jax.named_scope("wait_first"):
            ctail.wait()
            for cc in pend:
                cc.wait()
            pend = ()

        lo = lax.reduce_min(i_bufs[0][pl.ds(0, _L)], (0,))
        hi = lax.reduce_max(tail_v[...], (0,))
        loa = lax.shift_left(lax.shift_right_logical(lo, 4), 4)
        nvz = lax.shift_right_logical(hi - loa, 4) + 1

        with jax.named_scope("zero_acc"):
            @plsc.parallel_loop(0, nvz)
            def _(jv):
                for r in range(_L):
                    acc[pl.ds(r * pitch + loa + jv * _L, _L)] = zero

        lane_off = lax.iota(jnp.int32, _L) * pitch

        for j in range(_NSUB):
            b = j % 2
            with jax.named_scope(f"wait_{j}"):
                for cc in pend:
                    cc.wait()
            if j + 1 < _NSUB:
                pend = start(j + 1)
            else:
                pend = ()
            eb = e_bufs[b]
            ib = i_bufs[b]

            with jax.named_scope(f"scatter_{j}"):
                stride = nvec // _UNROLL

                @plsc.parallel_loop(0, stride)
                def _(v):
                    for q in range(_UNROLL):
                        sl = pl.ds((q * stride + v) * _L, _L)
                        plsc.addupdate_scatter(
                            acc, [ib[sl] + lane_off], eb[sl])

        with jax.named_scope("reduce"):
            @plsc.parallel_loop(0, nvz)
            def _(jv):
                t = acc[pl.ds(loa + jv * _L, _L)]
                for r in range(1, _L):
                    t = t + acc[pl.ds(r * pitch + loa + jv * _L, _L)]
                red[pl.ds(loa + jv * _L, _L)] = t

        with jax.named_scope("writeout"):
            pltpu.sync_copy(red, out_hbm.at[wid])

    return k(energy, idx)


def _tc_merge(partials, num_images):
    def body(p_ref, o_ref):
        o_ref[...] = jnp.sum(p_ref[...], axis=0)

    return pl.pallas_call(
        body,
        out_shape=jax.ShapeDtypeStruct((num_images,), jnp.float32),
    )(partials)


def kernel(atomic_energy, image_idx, n_atoms):
    n = atomic_energy.shape[0]
    num_images = n_atoms.shape[0]
    assert n % _NW == 0
    chunk = n // _NW

    idx32 = image_idx.astype(jnp.int32)
    energy = atomic_energy.astype(jnp.float32)

    partials = _sc_segment_partials(energy, idx32, num_images, chunk)
    return _tc_merge(partials, num_images)
